# baseline (speedup 1.0000x reference)
import functools

import jax
import jax.numpy as jnp
from jax import lax
from jax.experimental import pallas as pl
from jax.experimental.pallas import tpu as pltpu
from jax.experimental.pallas import tpu_sc as plsc

N = 10000
NPAD = 10240
E = 320000
D = 128
G = 64
GP = 128
NC = 2
NS = 16
NW = NC * NS
EPW = E // NW
ECH = 80
NCH = EPW // ECH
SCH = 40
SNCH = EPW // SCH
RPS = NPAD // NS
NPW = NPAD // NW
PCH = NPW // ECH
GRPS = GP // NS
TBLK = 2048


def _mesh():
    return plsc.VectorSubcoreMesh(
        core_axis_name="c", subcore_axis_name="s", num_cores=NC, num_subcores=NS
    )


def _deg_body(dst_hbm, ones_hbm, z1_hbm, out_hbm, idx_v, ones_v, acc_sh):
    c = lax.axis_index("c")
    s = lax.axis_index("s")
    wid = c * NS + s
    pltpu.sync_copy(z1_hbm, acc_sh.at[pl.ds(s * RPS, RPS)])
    pltpu.sync_copy(dst_hbm.at[wid], idx_v)
    pltpu.sync_copy(ones_hbm, ones_v)
    plsc.subcore_barrier()
    pltpu.sync_copy(ones_v, acc_sh.at[idx_v], add=True)
    plsc.subcore_barrier()
    pltpu.sync_copy(acc_sh.at[pl.ds(s * RPS, RPS)],
                    out_hbm.at[c, 0, pl.ds(s * RPS, RPS)])


def _deg_call(dstr, ones, z1):
    f = pl.kernel(
        _deg_body,
        out_type=jax.ShapeDtypeStruct((NC, 1, NPAD), jnp.float32),
        mesh=_mesh(),
        compiler_params=pltpu.CompilerParams(needs_layout_passes=False),
        scratch_types=[
            pltpu.VMEM((EPW,), jnp.int32),
            pltpu.VMEM((EPW,), jnp.float32),
            pltpu.VMEM_SHARED((NPAD,), jnp.float32),
        ],
    )
    return f(dstr, ones, z1)


def _scatter_body(g_hbm, src_hbm, dst_hbm, z2_hbm, out_hbm,
                  srcv, d0, d1, d2, d3, r0, r1, r2, r3, acc_sh,
                  sg0, sg1, sg2, sg3, sd0, sd1, sd2, sd3, sa0, sa1, sa2, sa3):
    c = lax.axis_index("c")
    s = lax.axis_index("s")
    wid = c * NS + s
    pltpu.sync_copy(z2_hbm, acc_sh.at[pl.ds(s * RPS, RPS)])
    pltpu.sync_copy(src_hbm.at[wid], srcv)
    plsc.subcore_barrier()

    sets = ((d0, r0, sg0, sd0, sa0), (d1, r1, sg1, sd1, sa1),
            (d2, r2, sg2, sd2, sa2), (d3, r3, sg3, sd3, sa3))
    last = SNCH - 1

    def didx(j, d, sem):
        pltpu.async_copy(dst_hbm.at[wid, j], d, sem)

    def gather(j, rows, sem):
        pltpu.async_copy(g_hbm.at[srcv.at[pl.ds(j * SCH, SCH)]], rows, sem)

    def wait_d(d, sem):
        pltpu.make_async_copy(dst_hbm.at[wid, 0], d, sem).wait()

    def wait_bytes(rows, sem):
        pltpu.make_async_copy(g_hbm.at[srcv.at[pl.ds(0, SCH)]], rows, sem).wait()

    for i, (d, r, sg, sd, sa) in enumerate(sets):
        didx(i, d, sd)
        gather(i, r, sg)

    def body(jj, carry):
        j = jj * 4
        for i, (d, r, sg, sd, sa) in enumerate(sets):
            wait_d(d, sd)
            wait_bytes(r, sg)
            pltpu.async_copy(r, acc_sh.at[d], sa, add=True)
        for i, (d, r, sg, sd, sa) in enumerate(sets):
            wait_bytes(r, sa)
            nxt = jnp.minimum(j + 4 + i, last)
            didx(nxt, d, sd)
            gather(nxt, r, sg)
        return carry

    lax.fori_loop(0, (SNCH - 2) // 4, body, 0)
    for i, (d, r, sg, sd, sa) in enumerate(sets):
        wait_d(d, sd)
        wait_bytes(r, sg)
        if i < 2:
            pltpu.sync_copy(r, acc_sh.at[d], add=True)
    plsc.subcore_barrier()
    pltpu.sync_copy(acc_sh.at[pl.ds(s * RPS, RPS)], out_hbm.at[c, pl.ds(s * RPS, RPS)])


def _scatter_call(g, srcr, dstr, z2):
    f = pl.kernel(
        _scatter_body,
        out_type=jax.ShapeDtypeStruct((NC, NPAD, D), jnp.float32),
        mesh=_mesh(),
        compiler_params=pltpu.CompilerParams(needs_layout_passes=False),
        scratch_types=(
            [pltpu.VMEM((EPW,), jnp.int32)]
            + [pltpu.VMEM((SCH,), jnp.int32)] * 4
            + [pltpu.VMEM((SCH, D), jnp.float32)] * 4
            + [pltpu.VMEM_SHARED((NPAD, D), jnp.float32)]
            + [pltpu.SemaphoreType.DMA] * 12
        ),
    )
    return f(g, srcr, dstr, z2)


def _pool_body(h2_hbm, bidx_hbm, ninf_hbm, zp_hbm,
               outmax_hbm, outsum_hbm,
               h2_v, bidx_v, maxacc_v, tbuf_v, mbuf_v, sum_sh, stage_sh):
    c = lax.axis_index("c")
    s = lax.axis_index("s")
    wid = c * NS + s
    pltpu.sync_copy(h2_hbm.at[pl.ds(wid * NPW, NPW)], h2_v)
    pltpu.sync_copy(bidx_hbm.at[wid], bidx_v)
    pltpu.sync_copy(ninf_hbm, maxacc_v)
    pltpu.sync_copy(zp_hbm, sum_sh.at[pl.ds(s * GRPS, GRPS)])
    plsc.subcore_barrier()

    pltpu.sync_copy(h2_v, sum_sh.at[bidx_v], add=True)

    lanes = lax.broadcasted_iota(jnp.int32, (16,), 0)

    def mbody(q, carry):
        bvec = bidx_v[pl.ds(q * 16, 16)]
        nvec = lanes + q * 16
        for t in range(D):
            colvec = (lanes * 8 + t) & (D - 1)
            hval = plsc.load_gather(h2_v, [nvec, colvec])
            cur = plsc.load_gather(maxacc_v, [bvec, colvec])
            plsc.store_scatter(maxacc_v, [bvec, colvec], jnp.maximum(cur, hval))
        return carry

    lax.fori_loop(0, NPW // 16, mbody, 0)

    pltpu.sync_copy(maxacc_v, stage_sh.at[s])
    plsc.subcore_barrier()
    pltpu.sync_copy(stage_sh.at[0, pl.ds(s * GRPS, GRPS)], mbuf_v)

    def cbody(t, carry):
        pltpu.sync_copy(stage_sh.at[t, pl.ds(s * GRPS, GRPS)], tbuf_v)
        for r in range(GRPS):
            for k in range(D // 16):
                sl = pl.ds(k * 16, 16)
                mbuf_v[r, sl] = jnp.maximum(mbuf_v[r, sl], tbuf_v[r, sl])
        return carry

    lax.fori_loop(1, NS, cbody, 0)
    pltpu.sync_copy(mbuf_v, outmax_hbm.at[c, pl.ds(s * GRPS, GRPS)])
    pltpu.sync_copy(sum_sh.at[pl.ds(s * GRPS, GRPS)],
                    outsum_hbm.at[c, pl.ds(s * GRPS, GRPS)])


def _pool_call(h2, bidxr, ninf, zp):
    f = pl.kernel(
        _pool_body,
        out_type=(
            jax.ShapeDtypeStruct((NC, GP, D), jnp.float32),
            jax.ShapeDtypeStruct((NC, GP, D), jnp.float32),
        ),
        mesh=_mesh(),
        compiler_params=pltpu.CompilerParams(needs_layout_passes=False),
        scratch_types=[
            pltpu.VMEM((NPW, D), jnp.float32),
            pltpu.VMEM((NPW,), jnp.int32),
            pltpu.VMEM((GP, D), jnp.float32),
            pltpu.VMEM((GRPS, D), jnp.float32),
            pltpu.VMEM((GRPS, D), jnp.float32),
            pltpu.VMEM_SHARED((GP, D), jnp.float32),
            pltpu.VMEM_SHARED((NS, GP, D), jnp.float32),
        ],
    )
    return f(h2, bidxr, ninf, zp)


def _dinv_body(dp_ref, o_ref):
    o_ref[...] = lax.rsqrt(dp_ref[0] + dp_ref[1] + 1.0)


def _dinv_call(degparts2d):
    return pl.pallas_call(
        _dinv_body,
        out_shape=jax.ShapeDtypeStruct((NPAD // D, D), jnp.float32),
    )(degparts2d)


def _mm_body(x_ref, w_ref, o_ref):
    o_ref[...] = jnp.dot(x_ref[...], w_ref[...],
                         preferred_element_type=jnp.float32)


def _mm_call(x_p, W):
    return pl.pallas_call(
        _mm_body,
        grid=(NPAD // TBLK,),
        in_specs=[
            pl.BlockSpec((TBLK, D), lambda i: (i, 0)),
            pl.BlockSpec((D, D), lambda i: (0, 0)),
        ],
        out_specs=pl.BlockSpec((TBLK, D), lambda i: (i, 0)),
        out_shape=jax.ShapeDtypeStruct((NPAD, D), jnp.float32),
    )(x_p, W)


def _scale_body(x_ref, dinv_ref, o_ref):
    o_ref[...] = x_ref[...] * dinv_ref[...]


def _scale_call(xw, dinvb):
    return pl.pallas_call(
        _scale_body,
        grid=(NPAD // TBLK,),
        in_specs=[
            pl.BlockSpec((TBLK, D), lambda i: (i, 0)),
            pl.BlockSpec((TBLK, D), lambda i: (i, 0)),
        ],
        out_specs=pl.BlockSpec((TBLK, D), lambda i: (i, 0)),
        out_shape=jax.ShapeDtypeStruct((NPAD, D), jnp.float32),
    )(xw, dinvb)


def _mm_scale_body(x_ref, w_ref, dinv_ref, o_ref):
    o_ref[...] = jnp.dot(x_ref[...], w_ref[...],
                         preferred_element_type=jnp.float32) * dinv_ref[...]


def _mm_scale_call(x_p, W, dinvb):
    return pl.pallas_call(
        _mm_scale_body,
        grid=(NPAD // TBLK,),
        in_specs=[
            pl.BlockSpec((TBLK, D), lambda i: (i, 0)),
            pl.BlockSpec((D, D), lambda i: (0, 0)),
            pl.BlockSpec((TBLK, D), lambda i: (i, 0)),
        ],
        out_specs=pl.BlockSpec((TBLK, D), lambda i: (i, 0)),
        out_shape=jax.ShapeDtypeStruct((NPAD, D), jnp.float32),
    )(x_p, W, dinvb)


def _combine_mm_body(sp_ref, g_ref, res_ref, dinv_ref, b_ref, w_ref, h_ref, g2_ref):
    t = (sp_ref[0] + sp_ref[1] + g_ref[...]) * dinv_ref[...] + b_ref[...] + res_ref[...]
    h = jnp.maximum(t, 0.0)
    h_ref[...] = h
    g2_ref[...] = jnp.dot(h, w_ref[...],
                          preferred_element_type=jnp.float32) * dinv_ref[...]


def _combine_mm_call(sparts, g, res, dinvb, b_row, W):
    return pl.pallas_call(
        _combine_mm_body,
        grid=(NPAD // TBLK,),
        in_specs=[
            pl.BlockSpec((NC, TBLK, D), lambda i: (0, i, 0)),
            pl.BlockSpec((TBLK, D), lambda i: (i, 0)),
            pl.BlockSpec((TBLK, D), lambda i: (i, 0)),
            pl.BlockSpec((TBLK, D), lambda i: (i, 0)),
            pl.BlockSpec((1, D), lambda i: (0, 0)),
            pl.BlockSpec((D, D), lambda i: (0, 0)),
        ],
        out_specs=[
            pl.BlockSpec((TBLK, D), lambda i: (i, 0)),
            pl.BlockSpec((TBLK, D), lambda i: (i, 0)),
        ],
        out_shape=[
            jax.ShapeDtypeStruct((NPAD, D), jnp.float32),
            jax.ShapeDtypeStruct((NPAD, D), jnp.float32),
        ],
    )(sparts, g, res, dinvb, b_row, W)


def _combine_body(sp_ref, g_ref, res_ref, dinv_ref, b_ref, o_ref):
    t = (sp_ref[0] + sp_ref[1] + g_ref[...]) * dinv_ref[...] + b_ref[...] + res_ref[...]
    o_ref[...] = jnp.maximum(t, 0.0)


def _combine_call(sparts, g, res, dinvb, b_row):
    return pl.pallas_call(
        _combine_body,
        grid=(NPAD // TBLK,),
        in_specs=[
            pl.BlockSpec((NC, TBLK, D), lambda i: (0, i, 0)),
            pl.BlockSpec((TBLK, D), lambda i: (i, 0)),
            pl.BlockSpec((TBLK, D), lambda i: (i, 0)),
            pl.BlockSpec((TBLK, D), lambda i: (i, 0)),
            pl.BlockSpec((1, D), lambda i: (0, 0)),
        ],
        out_specs=pl.BlockSpec((TBLK, D), lambda i: (i, 0)),
        out_shape=jax.ShapeDtypeStruct((NPAD, D), jnp.float32),
    )(sparts, g, res, dinvb, b_row)


def _head_body(mp_ref, sp_ref, mol_ref, molW_ref, molb_ref,
               w1a_ref, w1b_ref, w1c_ref, b1_ref, w2_ref, b2_ref, o_ref):
    m = jnp.maximum(mp_ref[0], mp_ref[1])[:G]
    a = (sp_ref[0] + sp_ref[1])[:G]
    mh = jnp.dot(mol_ref[...], molW_ref[...],
                 preferred_element_type=jnp.float32) + molb_ref[...]
    mh = jnp.maximum(mh * lax.rsqrt(jnp.float32(1.0 + 1e-5)), 0.0)
    z = (jnp.dot(m, w1a_ref[...], preferred_element_type=jnp.float32)
         + jnp.dot(a, w1b_ref[...], preferred_element_type=jnp.float32)
         + jnp.dot(mh, w1c_ref[...], preferred_element_type=jnp.float32)
         + b1_ref[...])
    z = jnp.maximum(z, 0.0)
    o_ref[...] = jnp.dot(z, w2_ref[...],
                         preferred_element_type=jnp.float32) + b2_ref[...]


def _head_call(mp, sp, mol, molW, molb_row, w1a, w1b, w1c, b1_row, w2p, b2p):
    return pl.pallas_call(
        _head_body,
        out_shape=jax.ShapeDtypeStruct((G, D), jnp.float32),
    )(mp, sp, mol, molW, molb_row, w1a, w1b, w1c, b1_row, w2p, b2p)


def kernel(x, edge_index, batch, molecule_features, W1, b1, W2, b2,
           mol_W, mol_b, mlp_W1, mlp_b1, mlp_W2, mlp_b2):
    f32 = jnp.float32
    x_p = jnp.pad(x, ((0, NPAD - N), (0, 0)))
    batch_p = jnp.pad(batch, (0, NPAD - N), constant_values=G)
    srcr = edge_index[0].reshape(NW, EPW)
    dstr40 = edge_index[1].reshape(NW, SNCH, SCH)
    dstrf = edge_index[1].reshape(NW, EPW)
    bidxr = batch_p.reshape(NW, NPW)
    ones = jnp.ones((EPW,), f32)
    z1 = jnp.zeros((RPS,), f32)
    z2 = jnp.zeros((RPS, D), f32)
    zp = jnp.zeros((GRPS, D), f32)
    ninf = jnp.full((GP, D), -jnp.inf, f32)

    xw1 = _mm_call(x_p, W1)
    degparts = _deg_call(dstrf, ones, z1)
    dinv2d = _dinv_call(degparts.reshape(NC, NPAD // D, D))
    dinvb = jnp.broadcast_to(dinv2d.reshape(NPAD, 1), (NPAD, D))

    g1 = _scale_call(xw1, dinvb)
    s1 = _scatter_call(g1, srcr, dstr40, z2)
    h, g2 = _combine_mm_call(s1, g1, x_p, dinvb, b1.reshape(1, D), W2)
    s2 = _scatter_call(g2, srcr, dstr40, z2)
    h2 = _combine_call(s2, g2, h, dinvb, b2.reshape(1, D))

    maxparts, sumparts = _pool_call(h2, bidxr, ninf, zp)

    w2p = jnp.zeros((D, D), f32).at[:, :1].set(mlp_W2)
    b2p = jnp.zeros((1, D), f32).at[0, :1].set(mlp_b2)
    outp = _head_call(maxparts, sumparts, molecule_features, mol_W,
                      mol_b.reshape(1, -1), mlp_W1[:D], mlp_W1[D:2 * D],
                      mlp_W1[2 * D:], mlp_b1.reshape(1, D), w2p, b2p)
    return outp[:, :1]


# speedup vs baseline: 1.0150x; 1.0150x over previous
import functools

import jax
import jax.numpy as jnp
from jax import lax
from jax.experimental import pallas as pl
from jax.experimental.pallas import tpu as pltpu
from jax.experimental.pallas import tpu_sc as plsc

N = 10000
NPAD = 10240
E = 320000
D = 128
G = 64
GP = 128
NC = 2
NS = 16
NW = NC * NS
EPW = E // NW
ECH = 80
NCH = EPW // ECH
SCH = 40
SNCH = EPW // SCH
RPS = NPAD // NS
NPW = NPAD // NW
PCH = NPW // ECH
GRPS = GP // NS
TBLK = 2048


def _mesh():
    return plsc.VectorSubcoreMesh(
        core_axis_name="c", subcore_axis_name="s", num_cores=NC, num_subcores=NS
    )


def _deg_body(dst_hbm, ones_hbm, z1_hbm, out_hbm, idx_v, ones_v, acc_sh):
    c = lax.axis_index("c")
    s = lax.axis_index("s")
    wid = c * NS + s
    pltpu.sync_copy(z1_hbm, acc_sh.at[pl.ds(s * RPS, RPS)])
    pltpu.sync_copy(dst_hbm.at[wid], idx_v)
    pltpu.sync_copy(ones_hbm, ones_v)
    plsc.subcore_barrier()
    pltpu.sync_copy(ones_v, acc_sh.at[idx_v], add=True)
    plsc.subcore_barrier()
    pltpu.sync_copy(acc_sh.at[pl.ds(s * RPS, RPS)],
                    out_hbm.at[c, 0, pl.ds(s * RPS, RPS)])


def _deg_call(dstr, ones, z1):
    f = pl.kernel(
        _deg_body,
        out_type=jax.ShapeDtypeStruct((NC, 1, NPAD), jnp.float32),
        mesh=_mesh(),
        compiler_params=pltpu.CompilerParams(needs_layout_passes=False),
        scratch_types=[
            pltpu.VMEM((EPW,), jnp.int32),
            pltpu.VMEM((EPW,), jnp.float32),
            pltpu.VMEM_SHARED((NPAD,), jnp.float32),
        ],
    )
    return f(dstr, ones, z1)


def _scatter_body(g_hbm, src_hbm, dst_hbm, z2_hbm, out_hbm,
                  srcv, d0, d1, d2, d3, r0, r1, r2, r3, acc_sh,
                  sg0, sg1, sg2, sg3, sd0, sd1, sd2, sd3, sa0, sa1, sa2, sa3):
    c = lax.axis_index("c")
    s = lax.axis_index("s")
    wid = c * NS + s
    pltpu.sync_copy(z2_hbm, acc_sh.at[pl.ds(s * RPS, RPS)])
    pltpu.sync_copy(src_hbm.at[wid], srcv)
    plsc.subcore_barrier()

    sets = ((d0, r0, sg0, sd0, sa0), (d1, r1, sg1, sd1, sa1),
            (d2, r2, sg2, sd2, sa2), (d3, r3, sg3, sd3, sa3))
    last = SNCH - 1

    def didx(j, d, sem):
        pltpu.async_copy(dst_hbm.at[wid, j], d, sem)

    def gather(j, rows, sem):
        pltpu.async_copy(g_hbm.at[srcv.at[pl.ds(j * SCH, SCH)]], rows, sem)

    def wait_d(d, sem):
        pltpu.make_async_copy(dst_hbm.at[wid, 0], d, sem).wait()

    def wait_bytes(rows, sem):
        pltpu.make_async_copy(g_hbm.at[srcv.at[pl.ds(0, SCH)]], rows, sem).wait()

    for i, (d, r, sg, sd, sa) in enumerate(sets):
        didx(i, d, sd)
        gather(i, r, sg)

    def body(jj, carry):
        j = jj * 4
        for i, (d, r, sg, sd, sa) in enumerate(sets):
            wait_d(d, sd)
            wait_bytes(r, sg)
            pltpu.async_copy(r, acc_sh.at[d], sa, add=True)
        for i, (d, r, sg, sd, sa) in enumerate(sets):
            wait_bytes(r, sa)
            nxt = jnp.minimum(j + 4 + i, last)
            didx(nxt, d, sd)
            gather(nxt, r, sg)
        return carry

    lax.fori_loop(0, (SNCH - 2) // 4, body, 0)
    for i, (d, r, sg, sd, sa) in enumerate(sets):
        wait_d(d, sd)
        wait_bytes(r, sg)
        if i < 2:
            pltpu.sync_copy(r, acc_sh.at[d], add=True)
    plsc.subcore_barrier()
    pltpu.sync_copy(acc_sh.at[pl.ds(s * RPS, RPS)], out_hbm.at[c, pl.ds(s * RPS, RPS)])


def _scatter_call(g, srcr, dstr, z2):
    f = pl.kernel(
        _scatter_body,
        out_type=jax.ShapeDtypeStruct((NC, NPAD, D), jnp.float32),
        mesh=_mesh(),
        compiler_params=pltpu.CompilerParams(needs_layout_passes=False),
        scratch_types=(
            [pltpu.VMEM((EPW,), jnp.int32)]
            + [pltpu.VMEM((SCH,), jnp.int32)] * 4
            + [pltpu.VMEM((SCH, D), jnp.float32)] * 4
            + [pltpu.VMEM_SHARED((NPAD, D), jnp.float32)]
            + [pltpu.SemaphoreType.DMA] * 12
        ),
    )
    return f(g, srcr, dstr, z2)


def _pool_body(h2_hbm, bidx_hbm, ninf_hbm, zp_hbm,
               outmax_hbm, outsum_hbm,
               h2_v, bidx_v, maxacc_v, tbuf_v, mbuf_v, sum_sh, stage_sh):
    c = lax.axis_index("c")
    s = lax.axis_index("s")
    wid = c * NS + s
    pltpu.sync_copy(h2_hbm.at[pl.ds(wid * NPW, NPW)], h2_v)
    pltpu.sync_copy(bidx_hbm.at[wid], bidx_v)
    pltpu.sync_copy(ninf_hbm, maxacc_v)
    pltpu.sync_copy(zp_hbm, sum_sh.at[pl.ds(s * GRPS, GRPS)])
    plsc.subcore_barrier()

    pltpu.sync_copy(h2_v, sum_sh.at[bidx_v], add=True)

    lanes = lax.broadcasted_iota(jnp.int32, (16,), 0)

    def mbody(q, carry):
        bvec = bidx_v[pl.ds(q * 16, 16)]
        nvec = lanes + q * 16
        for t in range(D):
            colvec = (lanes * 8 + t) & (D - 1)
            hval = plsc.load_gather(h2_v, [nvec, colvec])
            cur = plsc.load_gather(maxacc_v, [bvec, colvec])
            plsc.store_scatter(maxacc_v, [bvec, colvec], jnp.maximum(cur, hval))
        return carry

    lax.fori_loop(0, NPW // 16, mbody, 0)

    pltpu.sync_copy(maxacc_v, stage_sh.at[s])
    plsc.subcore_barrier()
    pltpu.sync_copy(stage_sh.at[0, pl.ds(s * GRPS, GRPS)], mbuf_v)

    def cbody(t, carry):
        pltpu.sync_copy(stage_sh.at[t, pl.ds(s * GRPS, GRPS)], tbuf_v)
        for r in range(GRPS):
            for k in range(D // 16):
                sl = pl.ds(k * 16, 16)
                mbuf_v[r, sl] = jnp.maximum(mbuf_v[r, sl], tbuf_v[r, sl])
        return carry

    lax.fori_loop(1, NS, cbody, 0)
    pltpu.sync_copy(mbuf_v, outmax_hbm.at[c, pl.ds(s * GRPS, GRPS)])
    pltpu.sync_copy(sum_sh.at[pl.ds(s * GRPS, GRPS)],
                    outsum_hbm.at[c, pl.ds(s * GRPS, GRPS)])


def _pool_call(h2, bidxr, ninf, zp):
    f = pl.kernel(
        _pool_body,
        out_type=(
            jax.ShapeDtypeStruct((NC, GP, D), jnp.float32),
            jax.ShapeDtypeStruct((NC, GP, D), jnp.float32),
        ),
        mesh=_mesh(),
        compiler_params=pltpu.CompilerParams(needs_layout_passes=False),
        scratch_types=[
            pltpu.VMEM((NPW, D), jnp.float32),
            pltpu.VMEM((NPW,), jnp.int32),
            pltpu.VMEM((GP, D), jnp.float32),
            pltpu.VMEM((GRPS, D), jnp.float32),
            pltpu.VMEM((GRPS, D), jnp.float32),
            pltpu.VMEM_SHARED((GP, D), jnp.float32),
            pltpu.VMEM_SHARED((NS, GP, D), jnp.float32),
        ],
    )
    return f(h2, bidxr, ninf, zp)


def _dinv_body(dp_ref, o_ref):
    o_ref[...] = lax.rsqrt(dp_ref[0] + dp_ref[1] + 1.0)


def _dinv_call(degparts2d):
    return pl.pallas_call(
        _dinv_body,
        out_shape=jax.ShapeDtypeStruct((NPAD // D, D), jnp.float32),
    )(degparts2d)


def _mm_scale_body(x_ref, w_ref, dinv_ref, o_ref):
    o_ref[...] = jnp.dot(x_ref[...], w_ref[...],
                         preferred_element_type=jnp.float32) * dinv_ref[...]


def _mm_scale_call(x_p, W, dinvb):
    return pl.pallas_call(
        _mm_scale_body,
        grid=(NPAD // TBLK,),
        in_specs=[
            pl.BlockSpec((TBLK, D), lambda i: (i, 0)),
            pl.BlockSpec((D, D), lambda i: (0, 0)),
            pl.BlockSpec((TBLK, D), lambda i: (i, 0)),
        ],
        out_specs=pl.BlockSpec((TBLK, D), lambda i: (i, 0)),
        out_shape=jax.ShapeDtypeStruct((NPAD, D), jnp.float32),
    )(x_p, W, dinvb)


def _combine_mm_body(sp_ref, g_ref, res_ref, dinv_ref, b_ref, w_ref, h_ref, g2_ref):
    t = (sp_ref[0] + sp_ref[1] + g_ref[...]) * dinv_ref[...] + b_ref[...] + res_ref[...]
    h = jnp.maximum(t, 0.0)
    h_ref[...] = h
    g2_ref[...] = jnp.dot(h, w_ref[...],
                          preferred_element_type=jnp.float32) * dinv_ref[...]


def _combine_mm_call(sparts, g, res, dinvb, b_row, W):
    return pl.pallas_call(
        _combine_mm_body,
        grid=(NPAD // TBLK,),
        in_specs=[
            pl.BlockSpec((NC, TBLK, D), lambda i: (0, i, 0)),
            pl.BlockSpec((TBLK, D), lambda i: (i, 0)),
            pl.BlockSpec((TBLK, D), lambda i: (i, 0)),
            pl.BlockSpec((TBLK, D), lambda i: (i, 0)),
            pl.BlockSpec((1, D), lambda i: (0, 0)),
            pl.BlockSpec((D, D), lambda i: (0, 0)),
        ],
        out_specs=[
            pl.BlockSpec((TBLK, D), lambda i: (i, 0)),
            pl.BlockSpec((TBLK, D), lambda i: (i, 0)),
        ],
        out_shape=[
            jax.ShapeDtypeStruct((NPAD, D), jnp.float32),
            jax.ShapeDtypeStruct((NPAD, D), jnp.float32),
        ],
    )(sparts, g, res, dinvb, b_row, W)


def _combine_body(sp_ref, g_ref, res_ref, dinv_ref, b_ref, o_ref):
    t = (sp_ref[0] + sp_ref[1] + g_ref[...]) * dinv_ref[...] + b_ref[...] + res_ref[...]
    o_ref[...] = jnp.maximum(t, 0.0)


def _combine_call(sparts, g, res, dinvb, b_row):
    return pl.pallas_call(
        _combine_body,
        grid=(NPAD // TBLK,),
        in_specs=[
            pl.BlockSpec((NC, TBLK, D), lambda i: (0, i, 0)),
            pl.BlockSpec((TBLK, D), lambda i: (i, 0)),
            pl.BlockSpec((TBLK, D), lambda i: (i, 0)),
            pl.BlockSpec((TBLK, D), lambda i: (i, 0)),
            pl.BlockSpec((1, D), lambda i: (0, 0)),
        ],
        out_specs=pl.BlockSpec((TBLK, D), lambda i: (i, 0)),
        out_shape=jax.ShapeDtypeStruct((NPAD, D), jnp.float32),
    )(sparts, g, res, dinvb, b_row)


def _head_body(mp_ref, sp_ref, mol_ref, molW_ref, molb_ref,
               w1a_ref, w1b_ref, w1c_ref, b1_ref, w2_ref, b2_ref, o_ref):
    m = jnp.maximum(mp_ref[0], mp_ref[1])[:G]
    a = (sp_ref[0] + sp_ref[1])[:G]
    mh = jnp.dot(mol_ref[...], molW_ref[...],
                 preferred_element_type=jnp.float32) + molb_ref[...]
    mh = jnp.maximum(mh * lax.rsqrt(jnp.float32(1.0 + 1e-5)), 0.0)
    z = (jnp.dot(m, w1a_ref[...], preferred_element_type=jnp.float32)
         + jnp.dot(a, w1b_ref[...], preferred_element_type=jnp.float32)
         + jnp.dot(mh, w1c_ref[...], preferred_element_type=jnp.float32)
         + b1_ref[...])
    z = jnp.maximum(z, 0.0)
    o_ref[...] = jnp.dot(z, w2_ref[...],
                         preferred_element_type=jnp.float32) + b2_ref[...]


def _head_call(mp, sp, mol, molW, molb_row, w1a, w1b, w1c, b1_row, w2p, b2p):
    return pl.pallas_call(
        _head_body,
        out_shape=jax.ShapeDtypeStruct((G, D), jnp.float32),
    )(mp, sp, mol, molW, molb_row, w1a, w1b, w1c, b1_row, w2p, b2p)


def kernel(x, edge_index, batch, molecule_features, W1, b1, W2, b2,
           mol_W, mol_b, mlp_W1, mlp_b1, mlp_W2, mlp_b2):
    f32 = jnp.float32
    x_p = jnp.pad(x, ((0, NPAD - N), (0, 0)))
    batch_p = jnp.pad(batch, (0, NPAD - N), constant_values=G)
    srcr = edge_index[0].reshape(NW, EPW)
    dstr40 = edge_index[1].reshape(NW, SNCH, SCH)
    dstrf = edge_index[1].reshape(NW, EPW)
    bidxr = batch_p.reshape(NW, NPW)
    ones = jnp.ones((EPW,), f32)
    z1 = jnp.zeros((RPS,), f32)
    z2 = jnp.zeros((RPS, D), f32)
    zp = jnp.zeros((GRPS, D), f32)
    ninf = jnp.full((GP, D), -jnp.inf, f32)

    degparts = _deg_call(dstrf, ones, z1)
    dinv2d = _dinv_call(degparts.reshape(NC, NPAD // D, D))
    dinvb = jnp.broadcast_to(dinv2d.reshape(NPAD, 1), (NPAD, D))

    g1 = _mm_scale_call(x_p, W1, dinvb)
    s1 = _scatter_call(g1, srcr, dstr40, z2)
    h, g2 = _combine_mm_call(s1, g1, x_p, dinvb, b1.reshape(1, D), W2)
    s2 = _scatter_call(g2, srcr, dstr40, z2)
    h2 = _combine_call(s2, g2, h, dinvb, b2.reshape(1, D))

    maxparts, sumparts = _pool_call(h2, bidxr, ninf, zp)

    w2p = jnp.zeros((D, D), f32).at[:, :1].set(mlp_W2)
    b2p = jnp.zeros((1, D), f32).at[0, :1].set(mlp_b2)
    outp = _head_call(maxparts, sumparts, molecule_features, mol_W,
                      mol_b.reshape(1, -1), mlp_W1[:D], mlp_W1[D:2 * D],
                      mlp_W1[2 * D:], mlp_b1.reshape(1, D), w2p, b2p)
    return outp[:, :1]
